# Initial kernel scaffold; baseline (speedup 1.0000x reference)
import jax
import jax.numpy as jnp
from jax.experimental import pallas as pl


def kernel(x, mask, feature_embedding, atse_embedding, atse_index_per_j, h_W1, h_b1, h_ln1_g, h_ln1_b, h_W2, h_b2, h_ln2_g, h_ln2_b, enc_W1, enc_b1, enc_W2, enc_b2):
    raise NotImplementedError("write your pallas kernel here")


# speedup vs baseline: 3.0307x; 3.0307x over previous
import functools

import jax
import jax.numpy as jnp
from jax import lax
from jax.experimental import pallas as pl
from jax.experimental.pallas import tpu as pltpu
from jax.experimental.pallas import tpu_sc as plsc

_EPS = 1e-5


def _sc_gather(table, idx, out_rows, row_w, num_cores, num_subcores):
    nw = num_cores * num_subcores
    per_w = out_rows // nw
    mesh = plsc.VectorSubcoreMesh(core_axis_name="c", subcore_axis_name="s")

    @functools.partial(
        pl.kernel,
        mesh=mesh,
        compiler_params=pltpu.CompilerParams(use_tc_tiling_on_sc=False),
        out_type=jax.ShapeDtypeStruct((out_rows, row_w), jnp.float32),
        scratch_types=[
            pltpu.VMEM((per_w,), jnp.int32),
            pltpu.VMEM((per_w, row_w), jnp.float32),
            pltpu.SemaphoreType.DMA,
        ],
    )
    def gather_kernel(table_hbm, idx_hbm, out_hbm, idx_v, rows_v, sem):
        wid = lax.axis_index("s") * num_cores + lax.axis_index("c")
        base = wid * per_w
        pltpu.sync_copy(idx_hbm.at[pl.ds(base, per_w)], idx_v)
        pltpu.async_copy(table_hbm.at[idx_v], rows_v, sem).wait()
        pltpu.sync_copy(rows_v, out_hbm.at[pl.ds(base, per_w)])

    return gather_kernel(table, idx)


def _ln_relu_cols(y, g, b):
    mu = jnp.mean(y, axis=0, keepdims=True)
    d = y - mu
    v = jnp.mean(d * d, axis=0, keepdims=True)
    return jnp.maximum(d * lax.rsqrt(v + _EPS) * g + b, 0.0)


def _ln_relu_rows(y):
    mu = jnp.mean(y, axis=1, keepdims=True)
    d = y - mu
    v = jnp.mean(d * d, axis=1, keepdims=True)
    return jnp.maximum(d * lax.rsqrt(v + _EPS), 0.0)


def _fused_body(nb, x_ref, m_ref, fT_ref, aeT_ref, w0_ref, w1f_ref, w1a_ref,
                b1_ref, g1_ref, bb1_ref, w2T_ref, b2_ref, g2_ref, bb2_ref,
                ew1_ref, eb1_ref, ew2_ref, eb2_ref,
                mu_ref, lv_ref, pooled_acc, cnt_acc):
    i = pl.program_id(0)
    n = pl.num_programs(0)

    @pl.when(i == 0)
    def _init():
        pooled_acc[...] = jnp.zeros_like(pooled_acc)
        cnt_acc[...] = jnp.zeros_like(cnt_acc)

    xb = x_ref[...]
    mb = m_ref[...]
    preT = (jnp.dot(w1f_ref[...], fT_ref[...], preferred_element_type=jnp.float32)
            + jnp.dot(w1a_ref[...], aeT_ref[...], preferred_element_type=jnp.float32)
            + b1_ref[...])
    w0 = w0_ref[...]
    g1 = g1_ref[...]
    bb1 = bb1_ref[...]
    w2T = w2T_ref[...]
    b2 = b2_ref[...]
    g2 = g2_ref[...]
    bb2 = bb2_ref[...]
    for b in range(nb):
        h1 = _ln_relu_cols(xb[b:b + 1, :] * w0 + preT, g1, bb1)
        h2 = jnp.dot(w2T, h1, preferred_element_type=jnp.float32) + b2
        h2 = _ln_relu_cols(h2, g2, bb2)
        pooled_acc[:, b:b + 1] += jnp.sum(h2 * mb[b:b + 1, :], axis=1,
                                          keepdims=True)
    cnt_acc[...] += jnp.sum(mb, axis=1, keepdims=True)

    @pl.when(i == n - 1)
    def _epilogue():
        eye = (lax.broadcasted_iota(jnp.int32, (nb, nb), 0)
               == lax.broadcasted_iota(jnp.int32, (nb, nb), 1)).astype(jnp.float32)
        pooled = lax.dot_general(eye, pooled_acc[...], (((1,), (1,)), ((), ())),
                                 preferred_element_type=jnp.float32)
        c = pooled / jnp.maximum(cnt_acc[...], 1.0)
        z = jnp.dot(c, ew1_ref[...], preferred_element_type=jnp.float32) + eb1_ref[...]
        z = _ln_relu_rows(z)
        o = jnp.dot(z, ew2_ref[...], preferred_element_type=jnp.float32) + eb2_ref[...]
        o = _ln_relu_rows(o)
        half = o.shape[1] // 2
        mu_ref[...] = o[:, :half]
        lv_ref[...] = o[:, half:]


def _run_fused(xp, mp, fTp, aeTp, w0, w1f, w1a, b1, g1, bb1, w2T, b2, g2, bb2,
               ew1, eb1, ew2, eb2, jb):
    nb, jp = xp.shape
    h = w0.shape[0]
    d = w2T.shape[0]
    ae = w1a.shape[1]
    he = ew1.shape[1]
    two_l = ew2.shape[1]
    grid = jp // jb

    def jmap(i):
        return (0, i)

    def cmap(i):
        return (0, 0)

    in_specs = [
        pl.BlockSpec((nb, jb), jmap),
        pl.BlockSpec((nb, jb), jmap),
        pl.BlockSpec((d, jb), jmap),
        pl.BlockSpec((ae, jb), jmap),
        pl.BlockSpec((h, 1), cmap),
        pl.BlockSpec((h, d), cmap),
        pl.BlockSpec((h, ae), cmap),
        pl.BlockSpec((h, 1), cmap),
        pl.BlockSpec((h, 1), cmap),
        pl.BlockSpec((h, 1), cmap),
        pl.BlockSpec((d, h), cmap),
        pl.BlockSpec((d, 1), cmap),
        pl.BlockSpec((d, 1), cmap),
        pl.BlockSpec((d, 1), cmap),
        pl.BlockSpec((d, he), cmap),
        pl.BlockSpec((nb, he), cmap),
        pl.BlockSpec((he, two_l), cmap),
        pl.BlockSpec((nb, two_l), cmap),
    ]
    out_specs = [
        pl.BlockSpec((nb, two_l // 2), cmap),
        pl.BlockSpec((nb, two_l // 2), cmap),
    ]
    out_shape = [
        jax.ShapeDtypeStruct((nb, two_l // 2), jnp.float32),
        jax.ShapeDtypeStruct((nb, two_l // 2), jnp.float32),
    ]
    return pl.pallas_call(
        functools.partial(_fused_body, nb),
        grid=(grid,),
        in_specs=in_specs,
        out_specs=out_specs,
        out_shape=out_shape,
        scratch_shapes=[
            pltpu.VMEM((d, nb), jnp.float32),
            pltpu.VMEM((nb, 1), jnp.float32),
        ],
    )(xp, mp, fTp, aeTp, w0, w1f, w1a, b1, g1, bb1, w2T, b2, g2, bb2,
      ew1, eb1, ew2, eb2)


def kernel(x, mask, feature_embedding, atse_embedding, atse_index_per_j,
           h_W1, h_b1, h_ln1_g, h_ln1_b, h_W2, h_b2, h_ln2_g, h_ln2_b,
           enc_W1, enc_b1, enc_W2, enc_b2):
    nb, j = x.shape
    d = feature_embedding.shape[1]

    info = plsc.get_sparse_core_info()
    nw = info.num_cores * info.num_subcores
    align = 8 * nw
    jp = ((j + align - 1) // align) * align
    pad = jp - j

    idx = jnp.pad(atse_index_per_j.astype(jnp.int32), (0, pad))
    ae_rows = _sc_gather(atse_embedding, idx, jp, atse_embedding.shape[1],
                         info.num_cores, info.num_subcores)

    xp = jnp.pad(x, ((0, 0), (0, pad)))
    mp = jnp.pad(mask.astype(jnp.float32), ((0, 0), (0, pad)))
    fTp = jnp.pad(feature_embedding.T, ((0, 0), (0, pad)))
    aeTp = ae_rows.T

    w1T = h_W1.T
    w0 = w1T[:, 0:1]
    w1f = w1T[:, 1:1 + d]
    w1a = w1T[:, 1 + d:]
    b1 = h_b1[:, None]
    g1 = h_ln1_g[:, None]
    bb1 = h_ln1_b[:, None]
    w2T = h_W2.T
    b2 = h_b2[:, None]
    g2 = h_ln2_g[:, None]
    bb2 = h_ln2_b[:, None]
    eb1 = jnp.broadcast_to(enc_b1[None, :], (nb, enc_b1.shape[0]))
    eb2 = jnp.broadcast_to(enc_b2[None, :], (nb, enc_b2.shape[0]))

    jb = jp // 8
    mu, lv = _run_fused(xp, mp, fTp, aeTp, w0, w1f, w1a, b1, g1, bb1,
                        w2T, b2, g2, bb2, enc_W1, eb1, enc_W2, eb2, jb)
    return (mu, lv)


# speedup vs baseline: 3.4428x; 1.1360x over previous
import functools

import jax
import jax.numpy as jnp
from jax import lax
from jax.experimental import pallas as pl
from jax.experimental.pallas import tpu as pltpu
from jax.experimental.pallas import tpu_sc as plsc

_EPS = 1e-5


def _sc_gather(table, idx, out_rows, row_w, num_cores, num_subcores):
    nw = num_cores * num_subcores
    per_w = out_rows // nw
    mesh = plsc.VectorSubcoreMesh(core_axis_name="c", subcore_axis_name="s")

    @functools.partial(
        pl.kernel,
        mesh=mesh,
        compiler_params=pltpu.CompilerParams(use_tc_tiling_on_sc=False),
        out_type=jax.ShapeDtypeStruct((out_rows, row_w), jnp.float32),
        scratch_types=[
            pltpu.VMEM((per_w,), jnp.int32),
            pltpu.VMEM((per_w, row_w), jnp.float32),
            pltpu.SemaphoreType.DMA,
        ],
    )
    def gather_kernel(table_hbm, idx_hbm, out_hbm, idx_v, rows_v, sem):
        wid = lax.axis_index("s") * num_cores + lax.axis_index("c")
        base = wid * per_w
        pltpu.sync_copy(idx_hbm.at[pl.ds(base, per_w)], idx_v)
        pltpu.async_copy(table_hbm.at[idx_v], rows_v, sem).wait()
        pltpu.sync_copy(rows_v, out_hbm.at[pl.ds(base, per_w)])

    return gather_kernel(table, idx)


def _ln_relu_rows(y):
    mu = jnp.mean(y, axis=1, keepdims=True)
    d = y - mu
    v = jnp.mean(d * d, axis=1, keepdims=True)
    return jnp.maximum(d * lax.rsqrt(v + _EPS), 0.0)


def _fused_body(nb, x_ref, m_ref, faT_ref, w0_ref, w1fa_ref,
                b1_ref, g1_ref, bb1_ref, w2blk_ref, b2p_ref, g2_ref, bb2_ref,
                ew1_ref, eb1_ref, ew2_ref, eb2_ref,
                mu_ref, lv_ref, pooled_acc, cnt_acc):
    i = pl.program_id(0)
    n = pl.num_programs(0)

    @pl.when(i == 0)
    def _init():
        pooled_acc[...] = jnp.zeros_like(pooled_acc)
        cnt_acc[...] = jnp.zeros_like(cnt_acc)

    xb = x_ref[...]
    mb = m_ref[...]
    w0 = w0_ref[...]
    g1 = g1_ref[...]
    bb1 = bb1_ref[...]

    preT = (jnp.dot(w1fa_ref[...], faT_ref[...],
                    preferred_element_type=jnp.float32) + b1_ref[...])

    h = preT.shape[0]
    inv_h = 1.0 / h
    mupre = jnp.sum(preT, axis=0, keepdims=True) * inv_h
    mwp = jnp.sum(w0 * preT, axis=0, keepdims=True) * inv_h
    mpp = jnp.sum(preT * preT, axis=0, keepdims=True) * inv_h
    mw0 = jnp.sum(w0) * inv_h
    mw0sq = jnp.sum(w0 * w0) * inv_h

    w2blk = w2blk_ref[...]
    b2p = b2p_ref[...]
    g2 = g2_ref[...][None, :, :]
    bb2 = bb2_ref[...][None, :, :]
    d = g2.shape[1]

    for p in range(nb // 2):
        halves = []
        for b in (2 * p, 2 * p + 1):
            xr = xb[b:b + 1, :]
            mu1 = xr * mw0 + mupre
            ey2 = (xr * xr) * mw0sq + (2.0 * xr) * mwp + mpp
            r = lax.rsqrt(jnp.maximum(ey2 - mu1 * mu1, 0.0) + _EPS)
            t = preT * r + w0 * (r * xr)
            t = t - r * mu1
            halves.append(jnp.maximum(g1 * t + bb1, 0.0))
        h1pair = jnp.concatenate(halves, axis=0)
        h2pair = (jnp.dot(w2blk, h1pair,
                          preferred_element_type=jnp.float32) + b2p)
        h23 = h2pair.reshape(2, d, h2pair.shape[1])
        m2 = jnp.mean(h23, axis=1, keepdims=True)
        d2 = h23 - m2
        v2 = jnp.mean(d2 * d2, axis=1, keepdims=True)
        h2n = jnp.maximum(d2 * lax.rsqrt(v2 + _EPS) * g2 + bb2, 0.0)
        mpair = mb[2 * p:2 * p + 2, :][:, None, :]
        pooled_acc[2 * p:2 * p + 2, :] += jnp.sum(h2n * mpair, axis=2)
    cnt_acc[...] += jnp.sum(mb, axis=1, keepdims=True)

    @pl.when(i == n - 1)
    def _epilogue():
        c = pooled_acc[...] / jnp.maximum(cnt_acc[...], 1.0)
        z = jnp.dot(c, ew1_ref[...], preferred_element_type=jnp.float32) + eb1_ref[...]
        z = _ln_relu_rows(z)
        o = jnp.dot(z, ew2_ref[...], preferred_element_type=jnp.float32) + eb2_ref[...]
        o = _ln_relu_rows(o)
        half = o.shape[1] // 2
        mu_ref[...] = o[:, :half]
        lv_ref[...] = o[:, half:]


def _run_fused(xp, mp, faTp, w0, w1fa, b1, g1, bb1, w2blk, b2p, g2, bb2,
               ew1, eb1, ew2, eb2, jb):
    nb, jp = xp.shape
    h = w0.shape[0]
    dae = w1fa.shape[1]
    d = g2.shape[0]
    he = ew1.shape[1]
    two_l = ew2.shape[1]
    grid = jp // jb

    def jmap(i):
        return (0, i)

    def cmap(i):
        return (0, 0)

    in_specs = [
        pl.BlockSpec((nb, jb), jmap),
        pl.BlockSpec((nb, jb), jmap),
        pl.BlockSpec((dae, jb), jmap),
        pl.BlockSpec((h, 1), cmap),
        pl.BlockSpec((h, dae), cmap),
        pl.BlockSpec((h, 1), cmap),
        pl.BlockSpec((h, 1), cmap),
        pl.BlockSpec((h, 1), cmap),
        pl.BlockSpec((2 * d, 2 * h), cmap),
        pl.BlockSpec((2 * d, 1), cmap),
        pl.BlockSpec((d, 1), cmap),
        pl.BlockSpec((d, 1), cmap),
        pl.BlockSpec((d, he), cmap),
        pl.BlockSpec((nb, he), cmap),
        pl.BlockSpec((he, two_l), cmap),
        pl.BlockSpec((nb, two_l), cmap),
    ]
    out_specs = [
        pl.BlockSpec((nb, two_l // 2), cmap),
        pl.BlockSpec((nb, two_l // 2), cmap),
    ]
    out_shape = [
        jax.ShapeDtypeStruct((nb, two_l // 2), jnp.float32),
        jax.ShapeDtypeStruct((nb, two_l // 2), jnp.float32),
    ]
    return pl.pallas_call(
        functools.partial(_fused_body, nb),
        grid=(grid,),
        in_specs=in_specs,
        out_specs=out_specs,
        out_shape=out_shape,
        scratch_shapes=[
            pltpu.VMEM((nb, d), jnp.float32),
            pltpu.VMEM((nb, 1), jnp.float32),
        ],
    )(xp, mp, faTp, w0, w1fa, b1, g1, bb1, w2blk, b2p, g2, bb2,
      ew1, eb1, ew2, eb2)


def kernel(x, mask, feature_embedding, atse_embedding, atse_index_per_j,
           h_W1, h_b1, h_ln1_g, h_ln1_b, h_W2, h_b2, h_ln2_g, h_ln2_b,
           enc_W1, enc_b1, enc_W2, enc_b2):
    nb, j = x.shape
    d = feature_embedding.shape[1]
    h = h_W1.shape[1]

    info = plsc.get_sparse_core_info()
    nw = info.num_cores * info.num_subcores
    align = 8 * nw
    jp = ((j + align - 1) // align) * align
    pad = jp - j

    idx = jnp.pad(atse_index_per_j.astype(jnp.int32), (0, pad))
    ae_rows = _sc_gather(atse_embedding, idx, jp, atse_embedding.shape[1],
                         info.num_cores, info.num_subcores)

    xp = jnp.pad(x, ((0, 0), (0, pad)))
    mp = jnp.pad(mask.astype(jnp.float32), ((0, 0), (0, pad)))
    faTp = jnp.concatenate(
        [jnp.pad(feature_embedding.T, ((0, 0), (0, pad))), ae_rows.T], axis=0)

    w1T = h_W1.T
    w0 = w1T[:, 0:1]
    w1fa = w1T[:, 1:]
    b1 = h_b1[:, None]
    g1 = h_ln1_g[:, None]
    bb1 = h_ln1_b[:, None]
    w2T = h_W2.T
    w2blk = jnp.zeros((2 * d, 2 * h), jnp.float32)
    w2blk = w2blk.at[:d, :h].set(w2T).at[d:, h:].set(w2T)
    b2p = jnp.concatenate([h_b2, h_b2])[:, None]
    g2 = h_ln2_g[:, None]
    bb2 = h_ln2_b[:, None]
    eb1 = jnp.broadcast_to(enc_b1[None, :], (nb, enc_b1.shape[0]))
    eb2 = jnp.broadcast_to(enc_b2[None, :], (nb, enc_b2.shape[0]))

    jb = jp // 8
    mu, lv = _run_fused(xp, mp, faTp, w0, w1fa, b1, g1, bb1, w2blk, b2p,
                        g2, bb2, enc_W1, eb1, enc_W2, eb2, jb)
    return (mu, lv)
